# baseline (speedup 1.0000x reference)
import functools

import numpy as np

import jax
import jax.numpy as jnp
from jax import lax
from jax.experimental import pallas as pl
from jax.experimental.pallas import tpu as pltpu
from jax.experimental.pallas import tpu_sc as plsc

_N = 10000
_D = 128
_E = 320000
_NEG = 0.2

_NC = 2
_NS = 16
_NW = _NC * _NS
_EPW = _E // _NW
_K = 80
_CH = _EPW // _K
_NPAD = 10240
_ZR = _NPAD // _NS
_ZC = _ZR // _K
_AR = _NPAD // 16
_APT = _AR // _NS


def _fc_body(x_ref, w_ref, al_ref, ar_ref, feat_ref, el_ref, er_ref):
    feat = jnp.dot(x_ref[...], w_ref[...], preferred_element_type=jnp.float32)
    feat_ref[...] = feat.astype(jnp.bfloat16)
    el_ref[...] = jnp.sum(feat * al_ref[...], axis=1)
    er_ref[...] = jnp.sum(feat * ar_ref[...], axis=1)


def _fc(x, W, attn_l, attn_r):
    return pl.pallas_call(
        _fc_body,
        out_shape=[
            jax.ShapeDtypeStruct((_N, _D), jnp.bfloat16),
            jax.ShapeDtypeStruct((_N,), jnp.float32),
            jax.ShapeDtypeStruct((_N,), jnp.float32),
        ],
    )(x, W, attn_l, attn_r)


def _edge_body(src_hbm, dst_hbm, el_hbm, er_hbm, feat_hbm,
               out_feat, out_p,
               src0, dst0, dsc0, p0, elg0, erg0, rows0, rowsb0,
               src1, dst1, dsc1, p1, elg1, erg1, rows1, rowsb1,
               asum_v, idt_v,
               acc_feat, acc_p,
               sem_idx, sem_gat, sem_sca0, sem_sca1):
    c = lax.axis_index("c")
    s = lax.axis_index("s")
    wid = s * _NC + c
    lane = lax.iota(jnp.int32, 16)
    bufs = (
        (src0, dst0, p0, elg0, erg0, rows0, sem_sca0, dsc0, rowsb0),
        (src1, dst1, p1, elg1, erg1, rows1, sem_sca1, dsc1, rowsb1),
    )

    def zero_rows(rows_v):
        def zrow(k, carry):
            z = jnp.zeros((16,), jnp.float32)
            for j in range(_D // 16):
                rows_v[k, pl.ds(j * 16, 16)] = z
            return carry

        lax.fori_loop(0, _K, zrow, 0)

    zero_rows(rows0)
    zero_rows(rows1)

    def zasum(k, carry):
        asum_v[k, :] = jnp.zeros((16,), jnp.float32)
        return carry

    lax.fori_loop(0, _AR, zasum, 0)
    for i in range(_AR // 128):
        for j in range(128 // 16):
            idt_v[i, pl.ds(j * 16, 16)] = i * 128 + j * 16 + lane
    for i in range(_ZC):
        r0 = s * _ZR + i * _K
        pltpu.sync_copy(rows0, acc_feat.at[pl.ds(r0, _K)])
    pltpu.sync_copy(
        asum_v.at[pl.ds(0, _APT)], acc_p.at[pl.ds(s * _APT, _APT)]
    )
    plsc.subcore_barrier()

    def issue_idx(ci, b):
        src_v, dst_v = bufs[b][0], bufs[b][1]
        base = jnp.minimum(wid * _EPW + ci * _K, _E - _K)
        pltpu.async_copy(src_hbm.at[pl.ds(base, _K)], src_v, sem_idx)
        pltpu.async_copy(dst_hbm.at[pl.ds(base, _K)], dst_v, sem_idx)

    def wait_idx(b):
        src_v, dst_v = bufs[b][0], bufs[b][1]
        pltpu.make_async_copy(src_hbm.at[pl.ds(0, _K)], src_v, sem_idx).wait()
        pltpu.make_async_copy(dst_hbm.at[pl.ds(0, _K)], dst_v, sem_idx).wait()

    def issue_gathers(b):
        src_v, dst_v, elg_v, erg_v, rowsb_v = (
            bufs[b][0], bufs[b][1], bufs[b][3], bufs[b][4], bufs[b][8]
        )
        pltpu.async_copy(feat_hbm.at[src_v], rowsb_v, sem_gat)
        pltpu.async_copy(el_hbm.at[src_v], elg_v, sem_gat)
        pltpu.async_copy(er_hbm.at[dst_v], erg_v, sem_gat)

    def wait_gathers(b):
        src_v, dst_v, elg_v, erg_v, rowsb_v = (
            bufs[b][0], bufs[b][1], bufs[b][3], bufs[b][4], bufs[b][8]
        )
        pltpu.make_async_copy(feat_hbm.at[src_v], rowsb_v, sem_gat).wait()
        pltpu.make_async_copy(el_hbm.at[src_v], elg_v, sem_gat).wait()
        pltpu.make_async_copy(er_hbm.at[dst_v], erg_v, sem_gat).wait()

    def issue_scatter(b):
        dst_v, rows_v, sem, dsc_v = (
            bufs[b][1], bufs[b][5], bufs[b][6], bufs[b][7]
        )
        for j in range(_K // 16):
            dsc_v[pl.ds(j * 16, 16)] = dst_v[pl.ds(j * 16, 16)]
        pltpu.async_copy(rows_v, acc_feat.at[dsc_v], sem, add=True)

    def wait_scatter(b):
        rows_v, sem, dsc_v = bufs[b][5], bufs[b][6], bufs[b][7]
        pltpu.make_async_copy(rows_v, acc_feat.at[dsc_v], sem).wait()

    def compute_p(b):
        dst_v, p_v, elg_v, erg_v = (
            bufs[b][1], bufs[b][2], bufs[b][3], bufs[b][4]
        )
        for j in range(_K // 16):
            e = elg_v[pl.ds(j * 16, 16)] + erg_v[pl.ds(j * 16, 16)]
            e = jnp.where(e >= 0.0, e, e * _NEG)
            p = jnp.exp(e)
            p_v[pl.ds(j * 16, 16)] = p
            di = dst_v[pl.ds(j * 16, 16)]
            plsc.addupdate_scatter(
                asum_v, [lax.shift_right_logical(di, 4), di & 15], p
            )

    def scale(b):
        p_v, rows_v, rowsb_v = bufs[b][2], bufs[b][5], bufs[b][8]

        @plsc.parallel_loop(0, _K, unroll=8)
        def body(k):
            pv = plsc.load_gather(p_v, [jnp.full((16,), 0, jnp.int32) + k])
            for j in range(_D // 32):
                v = rowsb_v[k, pl.ds(j * 32, 32)]
                lo, hi = plsc.unpack(v, format=plsc.PackFormat.INTERLEAVED)
                rows_v[k, pl.ds(j * 32, 16)] = lo * pv
                rows_v[k, pl.ds(j * 32 + 16, 16)] = hi * pv

    issue_idx(0, 0)
    issue_idx(1, 1)
    wait_idx(0)
    wait_idx(1)
    issue_scatter(1)
    issue_gathers(0)
    issue_idx(1, 1)

    def step(ci, b):
        wait_gathers(b)
        wait_idx(1 - b)
        wait_scatter(1 - b)
        issue_gathers(1 - b)
        compute_p(b)
        scale(b)
        issue_scatter(b)
        issue_idx(ci + 2, b)

    def pair(t, carry):
        step(2 * t, 0)
        step(2 * t + 1, 1)
        return carry

    lax.fori_loop(0, (_CH - 1) // 2, pair, 0)
    wait_gathers(0)
    wait_idx(1)
    wait_scatter(1)
    compute_p(0)
    scale(0)
    issue_scatter(0)
    wait_scatter(0)
    for i in range(_AR // 128):
        pltpu.sync_copy(
            asum_v.at[pl.ds(i * 128, 128)], acc_p.at[idt_v.at[i]], add=True
        )
    plsc.subcore_barrier()

    r0 = s * _ZR
    pltpu.sync_copy(
        acc_feat.at[pl.ds(r0, _ZR)], out_feat.at[c, pl.ds(r0, _ZR)]
    )
    pltpu.sync_copy(
        acc_p.at[pl.ds(s * _APT, _APT)], out_p.at[c, pl.ds(s * _APT, _APT)]
    )


def _edge(src, dst, el, er, feat):
    mesh = plsc.VectorSubcoreMesh(core_axis_name="c", subcore_axis_name="s")
    buf = [
        pltpu.VMEM((_K,), jnp.int32),
        pltpu.VMEM((_K,), jnp.int32),
        pltpu.VMEM((_K,), jnp.int32),
        pltpu.VMEM((_K,), jnp.float32),
        pltpu.VMEM((_K,), jnp.float32),
        pltpu.VMEM((_K,), jnp.float32),
        pltpu.VMEM((_K, _D), jnp.float32),
        pltpu.VMEM((_K, _D), jnp.bfloat16),
    ]
    f = functools.partial(
        pl.kernel,
        out_type=[
            jax.ShapeDtypeStruct((_NC, _NPAD, _D), jnp.float32),
            jax.ShapeDtypeStruct((_NC, _AR, 16), jnp.float32),
        ],
        mesh=mesh,
        compiler_params=pltpu.CompilerParams(
            use_tc_tiling_on_sc=False, needs_layout_passes=False
        ),
        scratch_types=[
            *buf,
            *buf,
            pltpu.VMEM((_AR, 16), jnp.float32),
            pltpu.VMEM((_AR // 128, 128), jnp.int32),
            pltpu.VMEM_SHARED((_NPAD, _D), jnp.float32),
            pltpu.VMEM_SHARED((_AR, 16), jnp.float32),
            pltpu.SemaphoreType.DMA,
            pltpu.SemaphoreType.DMA,
            pltpu.SemaphoreType.DMA,
            pltpu.SemaphoreType.DMA,
        ],
    )(_edge_body)
    return f(src, dst, el, er, feat)


def _fin_body(pf_ref, pp_ref, x_ref, b_ref, out_ref):
    ssum = pf_ref[0, : _N] + pf_ref[1, : _N]
    den = pp_ref[0, : _N] + pp_ref[1, : _N]
    den = jnp.where(den == 0.0, 1.0, den)
    out_ref[...] = ssum / den[:, None] + x_ref[...] + b_ref[...]


def _finalize(pf, pp, x, bias):
    return pl.pallas_call(
        _fin_body,
        out_shape=jax.ShapeDtypeStruct((_N, _D), jnp.float32),
    )(pf, pp, x, bias.reshape(1, _D))


_PERM = (
    np.arange(_D).reshape(_D // 32, 2, 16).swapaxes(1, 2).reshape(_D)
)


def kernel(x, edge_index, W, attn_l, attn_r, bias):
    featb, el, er = _fc(
        x, W[:, _PERM], attn_l[:, _PERM], attn_r[:, _PERM]
    )
    src = edge_index[0]
    dst = edge_index[1]
    pf, pp = _edge(src, dst, el, er, featb)
    return _finalize(pf, pp.reshape(_NC, _AR * 16), x, bias)


# speedup vs baseline: 1.0020x; 1.0020x over previous
import functools

import numpy as np

import jax
import jax.numpy as jnp
from jax import lax
from jax.experimental import pallas as pl
from jax.experimental.pallas import tpu as pltpu
from jax.experimental.pallas import tpu_sc as plsc

_N = 10000
_D = 128
_E = 320000
_NEG = 0.2

_NC = 2
_NS = 16
_NW = _NC * _NS
_EPW = _E // _NW
_K = 80
_CH = _EPW // _K
_NPAD = 10240
_ZR = _NPAD // _NS
_ZC = _ZR // _K
_AR = _NPAD // 16
_APT = _AR // _NS


def _fc_body(x_ref, w_ref, al_ref, ar_ref, feat_ref, el_ref, er_ref):
    feat = jnp.dot(x_ref[...], w_ref[...], preferred_element_type=jnp.float32)
    feat_ref[...] = feat.astype(jnp.bfloat16)
    el_ref[...] = jnp.sum(feat * al_ref[...], axis=1)
    er_ref[...] = jnp.sum(feat * ar_ref[...], axis=1)


def _fc(x, W, attn_l, attn_r):
    return pl.pallas_call(
        _fc_body,
        out_shape=[
            jax.ShapeDtypeStruct((_N, _D), jnp.bfloat16),
            jax.ShapeDtypeStruct((_N,), jnp.float32),
            jax.ShapeDtypeStruct((_N,), jnp.float32),
        ],
    )(x, W, attn_l, attn_r)


def _edge_body(src_hbm, dst_hbm, el_hbm, er_hbm, feat_hbm,
               out_feat, out_p,
               src0, dst0, dsc0, p0, elg0, erg0, rows0, rowsb0,
               src1, dst1, dsc1, p1, elg1, erg1, rows1, rowsb1,
               asum_v, idt_v,
               acc_feat, acc_p,
               sem_idx, sem_gat, sem_sca0, sem_sca1):
    c = lax.axis_index("c")
    s = lax.axis_index("s")
    wid = s * _NC + c
    lane = lax.iota(jnp.int32, 16)
    bufs = (
        (src0, dst0, p0, elg0, erg0, rows0, sem_sca0, dsc0, rowsb0),
        (src1, dst1, p1, elg1, erg1, rows1, sem_sca1, dsc1, rowsb1),
    )

    def zero_rows(rows_v):
        def zrow(k, carry):
            z = jnp.zeros((16,), jnp.float32)
            for j in range(_D // 16):
                rows_v[k, pl.ds(j * 16, 16)] = z
            return carry

        lax.fori_loop(0, _K, zrow, 0)

    zero_rows(rows0)
    zero_rows(rows1)

    def zasum(k, carry):
        asum_v[k, :] = jnp.zeros((16,), jnp.float32)
        return carry

    lax.fori_loop(0, _AR, zasum, 0)
    for i in range(_AR // 128):
        for j in range(128 // 16):
            idt_v[i, pl.ds(j * 16, 16)] = i * 128 + j * 16 + lane
    for i in range(_ZC):
        r0 = s * _ZR + i * _K
        pltpu.sync_copy(rows0, acc_feat.at[pl.ds(r0, _K)])
    pltpu.sync_copy(
        asum_v.at[pl.ds(0, _APT)], acc_p.at[pl.ds(s * _APT, _APT)]
    )
    plsc.subcore_barrier()

    def issue_idx(ci, b):
        src_v, dst_v = bufs[b][0], bufs[b][1]
        base = jnp.minimum(wid * _EPW + ci * _K, _E - _K)
        pltpu.async_copy(src_hbm.at[pl.ds(base, _K)], src_v, sem_idx)
        pltpu.async_copy(dst_hbm.at[pl.ds(base, _K)], dst_v, sem_idx)

    def wait_idx(b):
        src_v, dst_v = bufs[b][0], bufs[b][1]
        pltpu.make_async_copy(src_hbm.at[pl.ds(0, _K)], src_v, sem_idx).wait()
        pltpu.make_async_copy(dst_hbm.at[pl.ds(0, _K)], dst_v, sem_idx).wait()

    def issue_gathers(b):
        src_v, dst_v, elg_v, erg_v, rowsb_v = (
            bufs[b][0], bufs[b][1], bufs[b][3], bufs[b][4], bufs[b][8]
        )
        pltpu.async_copy(feat_hbm.at[src_v], rowsb_v, sem_gat)
        pltpu.async_copy(el_hbm.at[src_v], elg_v, sem_gat)
        pltpu.async_copy(er_hbm.at[dst_v], erg_v, sem_gat)

    def wait_gathers(b):
        src_v, dst_v, elg_v, erg_v, rowsb_v = (
            bufs[b][0], bufs[b][1], bufs[b][3], bufs[b][4], bufs[b][8]
        )
        pltpu.make_async_copy(feat_hbm.at[src_v], rowsb_v, sem_gat).wait()
        pltpu.make_async_copy(el_hbm.at[src_v], elg_v, sem_gat).wait()
        pltpu.make_async_copy(er_hbm.at[dst_v], erg_v, sem_gat).wait()

    def issue_scatter(b):
        dst_v, rows_v, sem, dsc_v = (
            bufs[b][1], bufs[b][5], bufs[b][6], bufs[b][7]
        )
        for j in range(_K // 16):
            dsc_v[pl.ds(j * 16, 16)] = dst_v[pl.ds(j * 16, 16)]
        pltpu.async_copy(rows_v, acc_feat.at[dsc_v], sem, add=True)

    def wait_scatter(b):
        rows_v, sem, dsc_v = bufs[b][5], bufs[b][6], bufs[b][7]
        pltpu.make_async_copy(rows_v, acc_feat.at[dsc_v], sem).wait()

    def compute_p(b):
        dst_v, p_v, elg_v, erg_v = (
            bufs[b][1], bufs[b][2], bufs[b][3], bufs[b][4]
        )
        for j in range(_K // 16):
            e = elg_v[pl.ds(j * 16, 16)] + erg_v[pl.ds(j * 16, 16)]
            e = jnp.where(e >= 0.0, e, e * _NEG)
            p = jnp.exp(e)
            p_v[pl.ds(j * 16, 16)] = p
            di = dst_v[pl.ds(j * 16, 16)]
            plsc.addupdate_scatter(
                asum_v, [lax.shift_right_logical(di, 4), di & 15], p
            )

    def scale(b):
        p_v, rows_v, rowsb_v = bufs[b][2], bufs[b][5], bufs[b][8]

        @plsc.parallel_loop(0, _K, unroll=4)
        def body(k):
            pv = plsc.load_gather(p_v, [jnp.full((16,), 0, jnp.int32) + k])
            for j in range(_D // 32):
                v = rowsb_v[k, pl.ds(j * 32, 32)]
                lo, hi = plsc.unpack(v, format=plsc.PackFormat.INTERLEAVED)
                rows_v[k, pl.ds(j * 32, 16)] = lo * pv
                rows_v[k, pl.ds(j * 32 + 16, 16)] = hi * pv

    issue_idx(0, 0)
    issue_idx(1, 1)
    wait_idx(0)
    wait_idx(1)
    issue_scatter(1)
    issue_gathers(0)
    issue_idx(1, 1)

    def step(ci, b):
        wait_gathers(b)
        wait_idx(1 - b)
        wait_scatter(1 - b)
        issue_gathers(1 - b)
        compute_p(b)
        scale(b)
        issue_scatter(b)
        issue_idx(ci + 2, b)

    def pair(t, carry):
        step(2 * t, 0)
        step(2 * t + 1, 1)
        return carry

    lax.fori_loop(0, (_CH - 1) // 2, pair, 0)
    wait_gathers(0)
    wait_idx(1)
    wait_scatter(1)
    compute_p(0)
    scale(0)
    issue_scatter(0)
    wait_scatter(0)
    for i in range(_AR // 128):
        pltpu.sync_copy(
            asum_v.at[pl.ds(i * 128, 128)], acc_p.at[idt_v.at[i]], add=True
        )
    plsc.subcore_barrier()

    r0 = s * _ZR
    pltpu.sync_copy(
        acc_feat.at[pl.ds(r0, _ZR)], out_feat.at[c, pl.ds(r0, _ZR)]
    )
    pltpu.sync_copy(
        acc_p.at[pl.ds(s * _APT, _APT)], out_p.at[c, pl.ds(s * _APT, _APT)]
    )


def _edge(src, dst, el, er, feat):
    mesh = plsc.VectorSubcoreMesh(core_axis_name="c", subcore_axis_name="s")
    buf = [
        pltpu.VMEM((_K,), jnp.int32),
        pltpu.VMEM((_K,), jnp.int32),
        pltpu.VMEM((_K,), jnp.int32),
        pltpu.VMEM((_K,), jnp.float32),
        pltpu.VMEM((_K,), jnp.float32),
        pltpu.VMEM((_K,), jnp.float32),
        pltpu.VMEM((_K, _D), jnp.float32),
        pltpu.VMEM((_K, _D), jnp.bfloat16),
    ]
    f = functools.partial(
        pl.kernel,
        out_type=[
            jax.ShapeDtypeStruct((_NC, _NPAD, _D), jnp.float32),
            jax.ShapeDtypeStruct((_NC, _AR, 16), jnp.float32),
        ],
        mesh=mesh,
        compiler_params=pltpu.CompilerParams(
            use_tc_tiling_on_sc=False, needs_layout_passes=False
        ),
        scratch_types=[
            *buf,
            *buf,
            pltpu.VMEM((_AR, 16), jnp.float32),
            pltpu.VMEM((_AR // 128, 128), jnp.int32),
            pltpu.VMEM_SHARED((_NPAD, _D), jnp.float32),
            pltpu.VMEM_SHARED((_AR, 16), jnp.float32),
            pltpu.SemaphoreType.DMA,
            pltpu.SemaphoreType.DMA,
            pltpu.SemaphoreType.DMA,
            pltpu.SemaphoreType.DMA,
        ],
    )(_edge_body)
    return f(src, dst, el, er, feat)


def _fin_body(pf_ref, pp_ref, x_ref, b_ref, out_ref):
    ssum = pf_ref[0, : _N] + pf_ref[1, : _N]
    den = pp_ref[0, : _N] + pp_ref[1, : _N]
    den = jnp.where(den == 0.0, 1.0, den)
    out_ref[...] = ssum / den[:, None] + x_ref[...] + b_ref[...]


def _finalize(pf, pp, x, bias):
    return pl.pallas_call(
        _fin_body,
        out_shape=jax.ShapeDtypeStruct((_N, _D), jnp.float32),
    )(pf, pp, x, bias.reshape(1, _D))


_PERM = (
    np.arange(_D).reshape(_D // 32, 2, 16).swapaxes(1, 2).reshape(_D)
)


def kernel(x, edge_index, W, attn_l, attn_r, bias):
    featb, el, er = _fc(
        x, W[:, _PERM], attn_l[:, _PERM], attn_r[:, _PERM]
    )
    src = edge_index[0]
    dst = edge_index[1]
    pf, pp = _edge(src, dst, el, er, featb)
    return _finalize(pf, pp.reshape(_NC, _AR * 16), x, bias)
